# baseline (speedup 1.0000x reference)
import functools

import jax
import jax.numpy as jnp
from jax import lax
from jax.experimental import pallas as pl
from jax.experimental.pallas import tpu as pltpu
from jax.experimental.pallas import tpu_sc as plsc

NUM_K = 1024
DIM = 64
NROWS = 16384
BLK = 1024
BETA_C = 0.25


def _dist_body(z_ref, e_ref, idx_ref, loss_ref, e2_ref):
    i = pl.program_id(0)
    z = z_ref[...]
    e = e_ref[...]

    @pl.when(i == 0)
    def _():
        e2_ref[...] = jnp.sum(e * e, axis=1)[None, :]

    z2 = jnp.sum(z * z, axis=1, keepdims=True)
    ks = lax.broadcasted_iota(jnp.int32, (1, NUM_K), 1).astype(jnp.float32)
    s = lax.dot_general(z, e, (((1,), (1,)), ((), ())),
                        preferred_element_type=jnp.float32)
    d = (z2 + e2_ref[...]) - 2.0 * s
    m = jnp.min(d, axis=1, keepdims=True)
    idx_f = jnp.min(jnp.where(d == m, ks, float(NUM_K)), axis=1, keepdims=True)
    idx_ref[...] = idx_f.astype(jnp.int32)

    @pl.when(i == 0)
    def _():
        loss_ref[...] = jnp.zeros_like(loss_ref)

    loss_ref[...] += jnp.sum(m).reshape(1, 1)


def _make_dist(nrows):
    return pl.pallas_call(
        _dist_body,
        grid=(nrows // BLK,),
        compiler_params=pltpu.CompilerParams(
            allow_input_fusion=[True, False]),
        in_specs=[
            pl.BlockSpec((BLK, DIM), lambda i: (i, 0)),
            pl.BlockSpec((NUM_K, DIM), lambda i: (0, 0)),
        ],
        out_specs=[
            pl.BlockSpec((BLK, 1), lambda i: (i, 0)),
            pl.BlockSpec((1, 1), lambda i: (0, 0)),
        ],
        out_shape=[
            jax.ShapeDtypeStruct((nrows, 1), jnp.int32),
            jax.ShapeDtypeStruct((1, 1), jnp.float32),
        ],
        scratch_shapes=[pltpu.VMEM((1, NUM_K), jnp.float32)],
    )


_SC_INFO = plsc.get_sparse_core_info()
_NW = _SC_INFO.num_cores * _SC_INFO.num_subcores
_CH = 128


def _make_gather(nrows):
    rows_w = nrows // _NW
    nch = rows_w // _CH
    mesh = plsc.VectorSubcoreMesh(core_axis_name="c", subcore_axis_name="s")

    @functools.partial(
        pl.kernel, mesh=mesh,
        compiler_params=pltpu.CompilerParams(use_tc_tiling_on_sc=False),
        out_type=jax.ShapeDtypeStruct((nrows, DIM), jnp.float32),
        scratch_types=[
            pltpu.VMEM((rows_w,), jnp.int32),
            pltpu.VMEM((rows_w, DIM), jnp.float32),
            pltpu.SemaphoreType.DMA,
            pltpu.SemaphoreType.DMA,
        ],
    )
    def gather_k(table_hbm, idx_hbm, out_hbm, idx_v, rows_v, sem, sem_o):
        wid = lax.axis_index("s") * _SC_INFO.num_cores + lax.axis_index("c")
        base = wid * rows_w
        pltpu.sync_copy(idx_hbm.at[pl.ds(base, rows_w)], idx_v)
        gathers = [
            pltpu.async_copy(
                table_hbm.at[idx_v.at[pl.ds(c * _CH, _CH)]],
                rows_v.at[pl.ds(c * _CH, _CH)], sem)
            for c in range(nch)
        ]
        writes = []
        for c in range(nch):
            gathers[c].wait()
            writes.append(pltpu.async_copy(
                rows_v.at[pl.ds(c * _CH, _CH)],
                out_hbm.at[pl.ds(base + c * _CH, _CH)], sem_o))
        for w in writes:
            w.wait()

    return gather_k


_dist_call = _make_dist(NROWS)
_gather_call = _make_gather(NROWS)


def kernel(z, embedding_weight):
    zp = jnp.transpose(z, (0, 2, 3, 1)).reshape(NROWS, DIM)
    idx2, loss_sum = _dist_call(zp, embedding_weight)
    idx = idx2.reshape(NROWS)
    zq = jnp.take(embedding_weight, idx, axis=0)
    zq_out = zq.reshape(16, 32, 32, DIM).transpose(0, 3, 1, 2)
    loss = loss_sum[0, 0] * (1.0 + BETA_C) / float(NROWS * DIM)
    return (zq_out, idx, loss)


# speedup vs baseline: 1.6086x; 1.6086x over previous
import functools

import jax
import jax.numpy as jnp
from jax import lax
from jax.experimental import pallas as pl
from jax.experimental.pallas import tpu as pltpu
from jax.experimental.pallas import tpu_sc as plsc

NUM_K = 1024
DIM = 64
NROWS = 16384
BLK = 1024
BETA_C = 0.25


def _dist_body(z_ref, e_ref, idx_ref, loss_ref, e2_ref):
    i = pl.program_id(0)
    z = z_ref[...]
    e = e_ref[...]

    @pl.when(i == 0)
    def _():
        e2_ref[...] = jnp.sum(e * e, axis=1)[None, :]

    z2 = jnp.sum(z * z, axis=1, keepdims=True)
    ks = lax.broadcasted_iota(jnp.int32, (1, NUM_K), 1).astype(jnp.float32)
    s = lax.dot_general(z, e, (((1,), (1,)), ((), ())),
                        preferred_element_type=jnp.float32)
    d = (z2 + e2_ref[...]) - 2.0 * s
    m = jnp.min(d, axis=1, keepdims=True)
    idx_f = jnp.min(jnp.where(d == m, ks, float(NUM_K)), axis=1, keepdims=True)
    idx_ref[...] = jnp.transpose(idx_f.astype(jnp.int32), (1, 0))[None]

    @pl.when(i == 0)
    def _():
        loss_ref[...] = jnp.zeros_like(loss_ref)

    loss_ref[...] += jnp.sum(m).reshape(1, 1)


def _make_dist(nrows):
    return pl.pallas_call(
        _dist_body,
        grid=(nrows // BLK,),
        compiler_params=pltpu.CompilerParams(
            allow_input_fusion=[True, False]),
        in_specs=[
            pl.BlockSpec((BLK, DIM), lambda i: (i, 0)),
            pl.BlockSpec((NUM_K, DIM), lambda i: (0, 0)),
        ],
        out_specs=[
            pl.BlockSpec((1, 1, BLK), lambda i: (i, 0, 0)),
            pl.BlockSpec((1, 1), lambda i: (0, 0)),
        ],
        out_shape=[
            jax.ShapeDtypeStruct((nrows // BLK, 1, BLK), jnp.int32),
            jax.ShapeDtypeStruct((1, 1), jnp.float32),
        ],
        scratch_shapes=[pltpu.VMEM((1, NUM_K), jnp.float32)],
    )


_SC_INFO = plsc.get_sparse_core_info()
_NW = _SC_INFO.num_cores * _SC_INFO.num_subcores
_CH = 128


def _make_gather(nrows):
    rows_w = nrows // _NW
    nch = rows_w // _CH
    mesh = plsc.VectorSubcoreMesh(core_axis_name="c", subcore_axis_name="s")

    @functools.partial(
        pl.kernel, mesh=mesh,
        compiler_params=pltpu.CompilerParams(use_tc_tiling_on_sc=False),
        out_type=jax.ShapeDtypeStruct((nrows, DIM), jnp.float32),
        scratch_types=[
            pltpu.VMEM((rows_w,), jnp.int32),
            pltpu.VMEM((rows_w, DIM), jnp.float32),
            pltpu.SemaphoreType.DMA,
            pltpu.SemaphoreType.DMA,
        ],
    )
    def gather_k(table_hbm, idx_hbm, out_hbm, idx_v, rows_v, sem, sem_o):
        wid = lax.axis_index("s") * _SC_INFO.num_cores + lax.axis_index("c")
        base = wid * rows_w
        pltpu.sync_copy(idx_hbm.at[pl.ds(base, rows_w)], idx_v)
        gathers = [
            pltpu.async_copy(
                table_hbm.at[idx_v.at[pl.ds(c * _CH, _CH)]],
                rows_v.at[pl.ds(c * _CH, _CH)], sem)
            for c in range(nch)
        ]
        writes = []
        for c in range(nch):
            gathers[c].wait()
            writes.append(pltpu.async_copy(
                rows_v.at[pl.ds(c * _CH, _CH)],
                out_hbm.at[pl.ds(base + c * _CH, _CH)], sem_o))
        for w in writes:
            w.wait()

    return gather_k


_dist_call = _make_dist(NROWS)
_gather_call = _make_gather(NROWS)


def kernel(z, embedding_weight):
    zp = jnp.transpose(z, (0, 2, 3, 1)).reshape(NROWS, DIM)
    idx3, loss_sum = _dist_call(zp, embedding_weight)
    idx = idx3.reshape(NROWS)
    zq = _gather_call(embedding_weight, idx)
    zq_out = zq.reshape(16, 32, 32, DIM).transpose(0, 3, 1, 2)
    loss = loss_sum[0, 0] * (1.0 + BETA_C) / float(NROWS * DIM)
    return (zq_out, idx, loss)


# speedup vs baseline: 1.7004x; 1.0571x over previous
import functools

import jax
import jax.numpy as jnp
from jax import lax
from jax.experimental import pallas as pl
from jax.experimental.pallas import tpu as pltpu
from jax.experimental.pallas import tpu_sc as plsc

NUM_K = 1024
DIM = 64
NROWS = 16384
BLK = 2048
BETA_C = 0.25


def _dist_body(z_ref, e_ref, idx_ref, loss_ref, e2_ref):
    i = pl.program_id(0)
    z = z_ref[...]
    e = e_ref[...]

    @pl.when(i == 0)
    def _():
        e2_ref[...] = jnp.sum(e * e, axis=1)[None, :]

    z2 = jnp.sum(z * z, axis=1, keepdims=True)
    ks = lax.broadcasted_iota(jnp.int32, (1, NUM_K), 1).astype(jnp.float32)
    s = lax.dot_general(z, e, (((1,), (1,)), ((), ())),
                        preferred_element_type=jnp.float32)
    d = (z2 + e2_ref[...]) - 2.0 * s
    m = jnp.min(d, axis=1, keepdims=True)
    idx_f = jnp.min(jnp.where(d == m, ks, float(NUM_K)), axis=1, keepdims=True)
    idx_ref[...] = jnp.transpose(idx_f.astype(jnp.int32), (1, 0))[None]

    @pl.when(i == 0)
    def _():
        loss_ref[...] = jnp.zeros_like(loss_ref)

    loss_ref[...] += jnp.sum(m).reshape(1, 1)


def _make_dist(nrows):
    return pl.pallas_call(
        _dist_body,
        grid=(nrows // BLK,),
        compiler_params=pltpu.CompilerParams(
            allow_input_fusion=[True, False]),
        in_specs=[
            pl.BlockSpec((BLK, DIM), lambda i: (i, 0)),
            pl.BlockSpec((NUM_K, DIM), lambda i: (0, 0)),
        ],
        out_specs=[
            pl.BlockSpec((1, 1, BLK), lambda i: (i, 0, 0)),
            pl.BlockSpec((1, 1), lambda i: (0, 0)),
        ],
        out_shape=[
            jax.ShapeDtypeStruct((nrows // BLK, 1, BLK), jnp.int32),
            jax.ShapeDtypeStruct((1, 1), jnp.float32),
        ],
        scratch_shapes=[pltpu.VMEM((1, NUM_K), jnp.float32)],
    )


_SC_INFO = plsc.get_sparse_core_info()
_NW = _SC_INFO.num_cores * _SC_INFO.num_subcores
_CH = 128


def _make_gather(nrows):
    rows_w = nrows // _NW
    nch = rows_w // _CH
    mesh = plsc.VectorSubcoreMesh(core_axis_name="c", subcore_axis_name="s")

    @functools.partial(
        pl.kernel, mesh=mesh,
        compiler_params=pltpu.CompilerParams(use_tc_tiling_on_sc=False),
        out_type=jax.ShapeDtypeStruct((nrows, DIM), jnp.float32),
        scratch_types=[
            pltpu.VMEM((rows_w,), jnp.int32),
            pltpu.VMEM((rows_w, DIM), jnp.float32),
            pltpu.SemaphoreType.DMA,
            pltpu.SemaphoreType.DMA,
        ],
    )
    def gather_k(table_hbm, idx_hbm, out_hbm, idx_v, rows_v, sem, sem_o):
        wid = lax.axis_index("s") * _SC_INFO.num_cores + lax.axis_index("c")
        base = wid * rows_w
        pltpu.sync_copy(idx_hbm.at[pl.ds(base, rows_w)], idx_v)
        gathers = [
            pltpu.async_copy(
                table_hbm.at[idx_v.at[pl.ds(c * _CH, _CH)]],
                rows_v.at[pl.ds(c * _CH, _CH)], sem)
            for c in range(nch)
        ]
        writes = []
        for c in range(nch):
            gathers[c].wait()
            writes.append(pltpu.async_copy(
                rows_v.at[pl.ds(c * _CH, _CH)],
                out_hbm.at[pl.ds(base + c * _CH, _CH)], sem_o))
        for w in writes:
            w.wait()

    return gather_k


_dist_call = _make_dist(NROWS)
_gather_call = _make_gather(NROWS)


def kernel(z, embedding_weight):
    zp = jnp.transpose(z, (0, 2, 3, 1)).reshape(NROWS, DIM)
    idx3, loss_sum = _dist_call(zp, embedding_weight)
    idx = idx3.reshape(NROWS)
    zq = _gather_call(embedding_weight, idx)
    zq_out = zq.reshape(16, 32, 32, DIM).transpose(0, 3, 1, 2)
    loss = loss_sum[0, 0] * (1.0 + BETA_C) / float(NROWS * DIM)
    return (zq_out, idx, loss)


# speedup vs baseline: 1.7452x; 1.0263x over previous
import functools

import jax
import jax.numpy as jnp
from jax import lax
from jax.experimental import pallas as pl
from jax.experimental.pallas import tpu as pltpu
from jax.experimental.pallas import tpu_sc as plsc

NUM_K = 1024
DIM = 64
NROWS = 16384
BLK = 4096
BETA_C = 0.25


def _dist_body(z_ref, e_ref, idx_ref, loss_ref, e2_ref):
    i = pl.program_id(0)
    z = z_ref[...]
    e = e_ref[...]

    @pl.when(i == 0)
    def _():
        e2_ref[...] = jnp.sum(e * e, axis=1)[None, :]

    z2 = jnp.sum(z * z, axis=1, keepdims=True)
    ks = lax.broadcasted_iota(jnp.int32, (1, NUM_K), 1).astype(jnp.float32)
    s = lax.dot_general(z, e, (((1,), (1,)), ((), ())),
                        preferred_element_type=jnp.float32)
    d = (z2 + e2_ref[...]) - 2.0 * s
    m = jnp.min(d, axis=1, keepdims=True)
    idx_f = jnp.min(jnp.where(d == m, ks, float(NUM_K)), axis=1, keepdims=True)
    idx_ref[...] = jnp.transpose(idx_f.astype(jnp.int32), (1, 0))[None]

    @pl.when(i == 0)
    def _():
        loss_ref[...] = jnp.zeros_like(loss_ref)

    loss_ref[...] += jnp.sum(m).reshape(1, 1)


def _make_dist(nrows):
    return pl.pallas_call(
        _dist_body,
        grid=(nrows // BLK,),
        compiler_params=pltpu.CompilerParams(
            allow_input_fusion=[True, False]),
        in_specs=[
            pl.BlockSpec((BLK, DIM), lambda i: (i, 0)),
            pl.BlockSpec((NUM_K, DIM), lambda i: (0, 0)),
        ],
        out_specs=[
            pl.BlockSpec((1, 1, BLK), lambda i: (i, 0, 0)),
            pl.BlockSpec((1, 1), lambda i: (0, 0)),
        ],
        out_shape=[
            jax.ShapeDtypeStruct((nrows // BLK, 1, BLK), jnp.int32),
            jax.ShapeDtypeStruct((1, 1), jnp.float32),
        ],
        scratch_shapes=[pltpu.VMEM((1, NUM_K), jnp.float32)],
    )


_SC_INFO = plsc.get_sparse_core_info()
_NW = _SC_INFO.num_cores * _SC_INFO.num_subcores
_CH = 128


def _make_gather(nrows):
    rows_w = nrows // _NW
    nch = rows_w // _CH
    mesh = plsc.VectorSubcoreMesh(core_axis_name="c", subcore_axis_name="s")

    @functools.partial(
        pl.kernel, mesh=mesh,
        compiler_params=pltpu.CompilerParams(use_tc_tiling_on_sc=False),
        out_type=jax.ShapeDtypeStruct((nrows, DIM), jnp.float32),
        scratch_types=[
            pltpu.VMEM((rows_w,), jnp.int32),
            pltpu.VMEM((rows_w, DIM), jnp.float32),
            pltpu.SemaphoreType.DMA,
            pltpu.SemaphoreType.DMA,
        ],
    )
    def gather_k(table_hbm, idx_hbm, out_hbm, idx_v, rows_v, sem, sem_o):
        wid = lax.axis_index("s") * _SC_INFO.num_cores + lax.axis_index("c")
        base = wid * rows_w
        pltpu.sync_copy(idx_hbm.at[pl.ds(base, rows_w)], idx_v)
        gathers = [
            pltpu.async_copy(
                table_hbm.at[idx_v.at[pl.ds(c * _CH, _CH)]],
                rows_v.at[pl.ds(c * _CH, _CH)], sem)
            for c in range(nch)
        ]
        writes = []
        for c in range(nch):
            gathers[c].wait()
            writes.append(pltpu.async_copy(
                rows_v.at[pl.ds(c * _CH, _CH)],
                out_hbm.at[pl.ds(base + c * _CH, _CH)], sem_o))
        for w in writes:
            w.wait()

    return gather_k


_dist_call = _make_dist(NROWS)
_gather_call = _make_gather(NROWS)


def kernel(z, embedding_weight):
    zp = jnp.transpose(z, (0, 2, 3, 1)).reshape(NROWS, DIM)
    idx3, loss_sum = _dist_call(zp, embedding_weight)
    idx = idx3.reshape(NROWS)
    zq = _gather_call(embedding_weight, idx)
    zq_out = zq.reshape(16, 32, 32, DIM).transpose(0, 3, 1, 2)
    loss = loss_sum[0, 0] * (1.0 + BETA_C) / float(NROWS * DIM)
    return (zq_out, idx, loss)


# speedup vs baseline: 1.7626x; 1.0100x over previous
import functools

import jax
import jax.numpy as jnp
from jax import lax
from jax.experimental import pallas as pl
from jax.experimental.pallas import tpu as pltpu
from jax.experimental.pallas import tpu_sc as plsc

NUM_K = 1024
DIM = 64
NROWS = 16384
BLK = 8192
BETA_C = 0.25


def _dist_body(z_ref, e_ref, idx_ref, loss_ref, e2_ref):
    i = pl.program_id(0)
    z = z_ref[...]
    e = e_ref[...]

    @pl.when(i == 0)
    def _():
        e2_ref[...] = jnp.sum(e * e, axis=1)[None, :]

    z2 = jnp.sum(z * z, axis=1, keepdims=True)
    ks = lax.broadcasted_iota(jnp.int32, (1, NUM_K), 1).astype(jnp.float32)
    s = lax.dot_general(z, e, (((1,), (1,)), ((), ())),
                        preferred_element_type=jnp.float32)
    d = (z2 + e2_ref[...]) - 2.0 * s
    m = jnp.min(d, axis=1, keepdims=True)
    idx_f = jnp.min(jnp.where(d == m, ks, float(NUM_K)), axis=1, keepdims=True)
    idx_ref[...] = jnp.transpose(idx_f.astype(jnp.int32), (1, 0))[None]

    @pl.when(i == 0)
    def _():
        loss_ref[...] = jnp.zeros_like(loss_ref)

    loss_ref[...] += jnp.sum(m).reshape(1, 1)


def _make_dist(nrows):
    return pl.pallas_call(
        _dist_body,
        grid=(nrows // BLK,),
        compiler_params=pltpu.CompilerParams(
            allow_input_fusion=[True, False]),
        in_specs=[
            pl.BlockSpec((BLK, DIM), lambda i: (i, 0)),
            pl.BlockSpec((NUM_K, DIM), lambda i: (0, 0)),
        ],
        out_specs=[
            pl.BlockSpec((1, 1, BLK), lambda i: (i, 0, 0)),
            pl.BlockSpec((1, 1), lambda i: (0, 0)),
        ],
        out_shape=[
            jax.ShapeDtypeStruct((nrows // BLK, 1, BLK), jnp.int32),
            jax.ShapeDtypeStruct((1, 1), jnp.float32),
        ],
        scratch_shapes=[pltpu.VMEM((1, NUM_K), jnp.float32)],
    )


_SC_INFO = plsc.get_sparse_core_info()
_NW = _SC_INFO.num_cores * _SC_INFO.num_subcores
_CH = 128


def _make_gather(nrows):
    rows_w = nrows // _NW
    nch = rows_w // _CH
    mesh = plsc.VectorSubcoreMesh(core_axis_name="c", subcore_axis_name="s")

    @functools.partial(
        pl.kernel, mesh=mesh,
        compiler_params=pltpu.CompilerParams(use_tc_tiling_on_sc=False),
        out_type=jax.ShapeDtypeStruct((nrows, DIM), jnp.float32),
        scratch_types=[
            pltpu.VMEM((rows_w,), jnp.int32),
            pltpu.VMEM((rows_w, DIM), jnp.float32),
            pltpu.SemaphoreType.DMA,
            pltpu.SemaphoreType.DMA,
        ],
    )
    def gather_k(table_hbm, idx_hbm, out_hbm, idx_v, rows_v, sem, sem_o):
        wid = lax.axis_index("s") * _SC_INFO.num_cores + lax.axis_index("c")
        base = wid * rows_w
        pltpu.sync_copy(idx_hbm.at[pl.ds(base, rows_w)], idx_v)
        gathers = [
            pltpu.async_copy(
                table_hbm.at[idx_v.at[pl.ds(c * _CH, _CH)]],
                rows_v.at[pl.ds(c * _CH, _CH)], sem)
            for c in range(nch)
        ]
        writes = []
        for c in range(nch):
            gathers[c].wait()
            writes.append(pltpu.async_copy(
                rows_v.at[pl.ds(c * _CH, _CH)],
                out_hbm.at[pl.ds(base + c * _CH, _CH)], sem_o))
        for w in writes:
            w.wait()

    return gather_k


_dist_call = _make_dist(NROWS)
_gather_call = _make_gather(NROWS)


def kernel(z, embedding_weight):
    zp = jnp.transpose(z, (0, 2, 3, 1)).reshape(NROWS, DIM)
    idx3, loss_sum = _dist_call(zp, embedding_weight)
    idx = idx3.reshape(NROWS)
    zq = _gather_call(embedding_weight, idx)
    zq_out = zq.reshape(16, 32, 32, DIM).transpose(0, 3, 1, 2)
    loss = loss_sum[0, 0] * (1.0 + BETA_C) / float(NROWS * DIM)
    return (zq_out, idx, loss)
